# baseline (speedup 1.0000x reference)
import functools

import jax
import jax.numpy as jnp
import numpy as np
from jax.experimental import pallas as pl
from jax.experimental.pallas import tpu as pltpu

_NVARS = 99
_SHAPE = (8, 40320, _NVARS)
_TOTAL = _SHAPE[0] * _SHAPE[1] * _SHAPE[2]
_LANES = 128
_ROWS = _TOTAL // _LANES
_PERIOD_ROWS = 99
_M = 24
_BR = _PERIOD_ROWS * _M
_GRID = _ROWS // _BR


def _affine_consts():
    scale = np.ones(_NVARS, dtype=np.float64)
    shift = np.zeros(_NVARS, dtype=np.float64)
    for i, m in zip((10, 11, 12), (11.0, 12.0, 13.0)):
        scale[i] = 1.0 / m
    i = np.arange(13, _NVARS).astype(np.float64)
    mu = (0.1 * i).astype(np.float32).astype(np.float64)
    sd = (1.0 + 0.01 * i).astype(np.float32).astype(np.float64)
    scale[13:] = 1.0 / sd
    shift[13:] = -(mu / sd)
    return scale.astype(np.float32), shift.astype(np.float32)


def _pattern_block():
    scale, shift = _affine_consts()
    idx = np.arange(_PERIOD_ROWS * _LANES) % _NVARS
    sc = np.tile(scale[idx].reshape(_PERIOD_ROWS, _LANES), (_M, 1))
    sh = np.tile(shift[idx].reshape(_PERIOD_ROWS, _LANES), (_M, 1))
    return jnp.asarray(sc), jnp.asarray(sh)


def _affine_body(x_ref, a_ref, b_ref, o_ref):
    o_ref[...] = x_ref[...] * a_ref[...] + b_ref[...]


@functools.partial(jax.jit)
def kernel(x):
    xf = x.reshape(_ROWS, _LANES)
    a, b = _pattern_block()
    out = pl.pallas_call(
        _affine_body,
        grid=(_GRID,),
        in_specs=[
            pl.BlockSpec((_BR, _LANES), lambda i: (i, 0)),
            pl.BlockSpec((_BR, _LANES), lambda i: (0, 0)),
            pl.BlockSpec((_BR, _LANES), lambda i: (0, 0)),
        ],
        out_specs=pl.BlockSpec((_BR, _LANES), lambda i: (i, 0)),
        out_shape=jax.ShapeDtypeStruct((_ROWS, _LANES), jnp.float32),
    )(xf, a, b)
    return out.reshape(_SHAPE)


# speedup vs baseline: 1.1030x; 1.1030x over previous
import functools

import jax
import jax.numpy as jnp
import numpy as np
from jax.experimental import pallas as pl
from jax.experimental.pallas import tpu as pltpu

_NVARS = 99
_SHAPE = (8, 40320, _NVARS)
_TOTAL = _SHAPE[0] * _SHAPE[1] * _SHAPE[2]
_PERIOD = 99 * 128
_ROWS = _TOTAL // _PERIOD
_BR = 40
_GRID = _ROWS // _BR


def _affine_consts():
    scale = np.ones(_NVARS, dtype=np.float64)
    shift = np.zeros(_NVARS, dtype=np.float64)
    for i, m in zip((10, 11, 12), (11.0, 12.0, 13.0)):
        scale[i] = 1.0 / m
    i = np.arange(13, _NVARS).astype(np.float64)
    mu = (0.1 * i).astype(np.float32).astype(np.float64)
    sd = (1.0 + 0.01 * i).astype(np.float32).astype(np.float64)
    scale[13:] = 1.0 / sd
    shift[13:] = -(mu / sd)
    return scale.astype(np.float32), shift.astype(np.float32)


def _pattern_row():
    scale, shift = _affine_consts()
    idx = np.arange(_PERIOD) % _NVARS
    return jnp.asarray(scale[idx][None, :]), jnp.asarray(shift[idx][None, :])


def _affine_body(x_ref, a_ref, b_ref, o_ref):
    o_ref[...] = x_ref[...] * a_ref[...] + b_ref[...]


@functools.partial(jax.jit)
def kernel(x):
    xf = x.reshape(_ROWS, _PERIOD)
    a, b = _pattern_row()
    out = pl.pallas_call(
        _affine_body,
        grid=(_GRID,),
        in_specs=[
            pl.BlockSpec((_BR, _PERIOD), lambda i: (i, 0)),
            pl.BlockSpec((1, _PERIOD), lambda i: (0, 0)),
            pl.BlockSpec((1, _PERIOD), lambda i: (0, 0)),
        ],
        out_specs=pl.BlockSpec((_BR, _PERIOD), lambda i: (i, 0)),
        out_shape=jax.ShapeDtypeStruct((_ROWS, _PERIOD), jnp.float32),
    )(xf, a, b)
    return out.reshape(_SHAPE)


# speedup vs baseline: 2.1762x; 1.9729x over previous
import functools

import jax
import jax.numpy as jnp
import numpy as np
from jax.experimental import pallas as pl
from jax.experimental.pallas import tpu as pltpu

_NVARS = 99
_SHAPE = (8, 40320, _NVARS)
_B1 = 1344
_GRID = _SHAPE[1] // _B1


def _affine_consts():
    scale = np.ones(_NVARS, dtype=np.float64)
    shift = np.zeros(_NVARS, dtype=np.float64)
    for i, m in zip((10, 11, 12), (11.0, 12.0, 13.0)):
        scale[i] = 1.0 / m
    i = np.arange(13, _NVARS).astype(np.float64)
    mu = (0.1 * i).astype(np.float32).astype(np.float64)
    sd = (1.0 + 0.01 * i).astype(np.float32).astype(np.float64)
    scale[13:] = 1.0 / sd
    shift[13:] = -(mu / sd)
    return scale.astype(np.float32), shift.astype(np.float32)


def _affine_body(x_ref, a_ref, b_ref, o_ref):
    o_ref[...] = x_ref[...] * a_ref[...] + b_ref[...]


@functools.partial(jax.jit)
def kernel(x):
    scale, shift = _affine_consts()
    a = jnp.asarray(scale)[None, None, :]
    b = jnp.asarray(shift)[None, None, :]
    return pl.pallas_call(
        _affine_body,
        grid=(_GRID,),
        in_specs=[
            pl.BlockSpec((_SHAPE[0], _B1, _NVARS), lambda i: (0, i, 0)),
            pl.BlockSpec((1, 1, _NVARS), lambda i: (0, 0, 0)),
            pl.BlockSpec((1, 1, _NVARS), lambda i: (0, 0, 0)),
        ],
        out_specs=pl.BlockSpec((_SHAPE[0], _B1, _NVARS), lambda i: (0, i, 0)),
        out_shape=jax.ShapeDtypeStruct(_SHAPE, jnp.float32),
    )(x, a, b)
